# baseline (device time: 748222 ns/iter reference)
import jax
import jax.numpy as jnp
from jax import lax
from jax.experimental import pallas as pl
from jax.experimental.pallas import tpu as pltpu

N_DEV = 16
WIRE_DTYPE = jnp.bfloat16


def kernel(x, w_mat, scale_x, scale_w):
    m, k = x.shape
    _, n = w_mat.shape
    m_per = m // N_DEV

    def body(x_ref, w_ref, sx_ref, sw_ref, out_ref,
             send_buf, recv_buf, send_sems, recv_sems, credit_sem):
        p = lax.axis_index("i")
        left = lax.rem(p - 1 + N_DEV, N_DEV)
        right = lax.rem(p + 1, N_DEV)

        barrier = pltpu.get_barrier_semaphore()
        for nbr in (left, right):
            pl.semaphore_signal(barrier, inc=1, device_id=(nbr,),
                                device_id_type=pl.DeviceIdType.MESH)
        pl.semaphore_wait(barrier, 2)

        def chunk_dot(c):
            xc = x_ref[pl.ds(c * m_per, m_per), :].astype(jnp.bfloat16)
            return lax.dot_general(
                xc, w_ref[...].astype(jnp.bfloat16),
                (((1,), (0,)), ((), ())),
                preferred_element_type=jnp.float32)

        prev_rdma = None
        for s in range(N_DEV - 1):
            c = lax.rem(p - 1 - s + 2 * N_DEV, N_DEV)
            part = chunk_dot(c)
            if s > 0:
                prev_rdma.wait_recv()
                part = part + recv_buf[(s - 1) % 2].astype(jnp.float32)
            if s >= 2:
                pl.semaphore_wait(credit_sem, 1)
            send_buf[s % 2] = part.astype(WIRE_DTYPE)
            if 1 <= s <= N_DEV - 3:
                pl.semaphore_signal(credit_sem, inc=1, device_id=(left,),
                                    device_id_type=pl.DeviceIdType.MESH)
            rdma = pltpu.make_async_remote_copy(
                src_ref=send_buf.at[s % 2],
                dst_ref=recv_buf.at[s % 2],
                send_sem=send_sems.at[s % 2],
                recv_sem=recv_sems.at[s % 2],
                device_id=(right,),
                device_id_type=pl.DeviceIdType.MESH)
            rdma.start()
            rdma.wait_send()
            prev_rdma = rdma

        part = chunk_dot(p)
        prev_rdma.wait_recv()
        total = part + recv_buf[(N_DEV - 2) % 2].astype(jnp.float32)
        y = total * (sx_ref[0] * sw_ref[0])
        out_ref[...] = y * (1.0 / (1.0 + jnp.exp(-jnp.clip(y, -60.0, 60.0))))

    return pl.pallas_call(
        body,
        out_shape=jax.ShapeDtypeStruct((m_per, n), jnp.float32),
        in_specs=[
            pl.BlockSpec(memory_space=pltpu.VMEM),
            pl.BlockSpec(memory_space=pltpu.VMEM),
            pl.BlockSpec(memory_space=pltpu.SMEM),
            pl.BlockSpec(memory_space=pltpu.SMEM),
        ],
        out_specs=pl.BlockSpec(memory_space=pltpu.VMEM),
        scratch_shapes=[
            pltpu.VMEM((2, m_per, n), WIRE_DTYPE),
            pltpu.VMEM((2, m_per, n), WIRE_DTYPE),
            pltpu.SemaphoreType.DMA((2,)),
            pltpu.SemaphoreType.DMA((2,)),
            pltpu.SemaphoreType.REGULAR,
        ],
        compiler_params=pltpu.CompilerParams(collective_id=0),
    )(x, w_mat, scale_x, scale_w)


# device time: 425024 ns/iter; 1.7604x vs baseline; 1.7604x over previous
import jax
import jax.numpy as jnp
from jax import lax
from jax.experimental import pallas as pl
from jax.experimental.pallas import tpu as pltpu

N_DEV = 16
WIRE_DTYPE = jnp.bfloat16


def kernel(x, w_mat, scale_x, scale_w):
    m, k = x.shape
    _, n = w_mat.shape
    m_per = m // N_DEV
    nh = n // 2

    def body(x_ref, w_ref, sx_ref, sw_ref, out_ref,
             send_cw, recv_cw, send_ccw, recv_ccw,
             ssem_cw, rsem_cw, ssem_ccw, rsem_ccw,
             credit_cw, credit_ccw):
        p = lax.axis_index("i")
        left = lax.rem(p - 1 + N_DEV, N_DEV)
        right = lax.rem(p + 1, N_DEV)

        barrier = pltpu.get_barrier_semaphore()
        for nbr in (left, right):
            pl.semaphore_signal(barrier, inc=1, device_id=(nbr,),
                                device_id_type=pl.DeviceIdType.MESH)
        pl.semaphore_wait(barrier, 2)

        def chunk_dot(c, col0):
            xc = x_ref[pl.ds(c * m_per, m_per), :].astype(jnp.bfloat16)
            wc = w_ref[:, pl.ds(col0, nh)].astype(jnp.bfloat16)
            return lax.dot_general(xc, wc, (((1,), (0,)), ((), ())),
                                   preferred_element_type=jnp.float32)

        cw_rdmas, ccw_rdmas = [], []
        for s in range(N_DEV - 1):
            c1 = lax.rem(p - 1 - s + 2 * N_DEV, N_DEV)
            c2 = lax.rem(p + 1 + s, N_DEV)
            part1 = chunk_dot(c1, 0)
            part2 = chunk_dot(c2, nh)
            if s > 0:
                cw_rdmas[s - 1].wait_recv()
                part1 = part1 + recv_cw[(s - 1) % 2].astype(jnp.float32)
                ccw_rdmas[s - 1].wait_recv()
                part2 = part2 + recv_ccw[(s - 1) % 2].astype(jnp.float32)
            if s >= 2:
                pl.semaphore_wait(credit_cw, 1)
                pl.semaphore_wait(credit_ccw, 1)
                cw_rdmas[s - 2].wait_send()
                ccw_rdmas[s - 2].wait_send()
            send_cw[s % 2] = part1.astype(WIRE_DTYPE)
            send_ccw[s % 2] = part2.astype(WIRE_DTYPE)
            if 1 <= s <= N_DEV - 3:
                pl.semaphore_signal(credit_cw, inc=1, device_id=(left,),
                                    device_id_type=pl.DeviceIdType.MESH)
                pl.semaphore_signal(credit_ccw, inc=1, device_id=(right,),
                                    device_id_type=pl.DeviceIdType.MESH)
            rdma1 = pltpu.make_async_remote_copy(
                src_ref=send_cw.at[s % 2], dst_ref=recv_cw.at[s % 2],
                send_sem=ssem_cw.at[s % 2], recv_sem=rsem_cw.at[s % 2],
                device_id=(right,), device_id_type=pl.DeviceIdType.MESH)
            rdma2 = pltpu.make_async_remote_copy(
                src_ref=send_ccw.at[s % 2], dst_ref=recv_ccw.at[s % 2],
                send_sem=ssem_ccw.at[s % 2], recv_sem=rsem_ccw.at[s % 2],
                device_id=(left,), device_id_type=pl.DeviceIdType.MESH)
            rdma1.start()
            rdma2.start()
            cw_rdmas.append(rdma1)
            ccw_rdmas.append(rdma2)

        part1 = chunk_dot(p, 0)
        part2 = chunk_dot(p, nh)
        scale = sx_ref[0] * sw_ref[0]
        last = (N_DEV - 2) % 2

        cw_rdmas[N_DEV - 2].wait_recv()
        y1 = (part1 + recv_cw[last].astype(jnp.float32)) * scale
        out_ref[:, :nh] = y1 * (1.0 / (1.0 + jnp.exp(-jnp.clip(y1, -60.0, 60.0))))
        ccw_rdmas[N_DEV - 2].wait_recv()
        y2 = (part2 + recv_ccw[last].astype(jnp.float32)) * scale
        out_ref[:, nh:] = y2 * (1.0 / (1.0 + jnp.exp(-jnp.clip(y2, -60.0, 60.0))))

        for r in (cw_rdmas[N_DEV - 3], ccw_rdmas[N_DEV - 3],
                  cw_rdmas[N_DEV - 2], ccw_rdmas[N_DEV - 2]):
            r.wait_send()

    return pl.pallas_call(
        body,
        out_shape=jax.ShapeDtypeStruct((m_per, n), jnp.float32),
        in_specs=[
            pl.BlockSpec(memory_space=pltpu.VMEM),
            pl.BlockSpec(memory_space=pltpu.VMEM),
            pl.BlockSpec(memory_space=pltpu.SMEM),
            pl.BlockSpec(memory_space=pltpu.SMEM),
        ],
        out_specs=pl.BlockSpec(memory_space=pltpu.VMEM),
        scratch_shapes=[
            pltpu.VMEM((2, m_per, nh), WIRE_DTYPE),
            pltpu.VMEM((2, m_per, nh), WIRE_DTYPE),
            pltpu.VMEM((2, m_per, nh), WIRE_DTYPE),
            pltpu.VMEM((2, m_per, nh), WIRE_DTYPE),
            pltpu.SemaphoreType.DMA((2,)),
            pltpu.SemaphoreType.DMA((2,)),
            pltpu.SemaphoreType.DMA((2,)),
            pltpu.SemaphoreType.DMA((2,)),
            pltpu.SemaphoreType.REGULAR,
            pltpu.SemaphoreType.REGULAR,
        ],
        compiler_params=pltpu.CompilerParams(collective_id=0),
    )(x, w_mat, scale_x, scale_w)


# device time: 356706 ns/iter; 2.0976x vs baseline; 1.1915x over previous
import jax
import jax.numpy as jnp
from jax import lax
from jax.experimental import pallas as pl
from jax.experimental.pallas import tpu as pltpu

N_DEV = 16
N_RINGS = 4
WIRE_DTYPE = jnp.bfloat16


def kernel(x, w_mat, scale_x, scale_w):
    m, k = x.shape
    _, n = w_mat.shape
    m_per = m // N_DEV
    nq = n // N_RINGS

    rings = [(True, 0), (False, 2 * nq), (True, nq), (False, 3 * nq)]

    def body(x_ref, w_ref, sx_ref, sw_ref, out_ref, *scratch):
        send_bufs = scratch[0:4]
        recv_bufs = scratch[4:8]
        ssems = scratch[8:12]
        rsems = scratch[12:16]
        credits = scratch[16:20]

        p = lax.axis_index("i")
        left = lax.rem(p - 1 + N_DEV, N_DEV)
        right = lax.rem(p + 1, N_DEV)

        barrier = pltpu.get_barrier_semaphore()
        for nbr in (left, right):
            pl.semaphore_signal(barrier, inc=1, device_id=(nbr,),
                                device_id_type=pl.DeviceIdType.MESH)
        pl.semaphore_wait(barrier, 2)

        def chunk_dot(c, col0):
            xc = x_ref[pl.ds(c * m_per, m_per), :].astype(jnp.bfloat16)
            wc = w_ref[:, pl.ds(col0, nq)].astype(jnp.bfloat16)
            return lax.dot_general(xc, wc, (((1,), (0,)), ((), ())),
                                   preferred_element_type=jnp.float32)

        rdmas = [[] for _ in rings]
        for s in range(N_DEV - 1):
            c_cw = lax.rem(p - 1 - s + 2 * N_DEV, N_DEV)
            c_ccw = lax.rem(p + 1 + s, N_DEV)
            parts = [chunk_dot(c_cw if is_cw else c_ccw, col0)
                     for is_cw, col0 in rings]
            for q, (is_cw, col0) in enumerate(rings):
                part = parts[q]
                if s > 0:
                    rdmas[q][s - 1].wait_recv()
                    part = part + recv_bufs[q][(s - 1) % 2].astype(jnp.float32)
                if s >= 2:
                    pl.semaphore_wait(credits[q], 1)
                    rdmas[q][s - 2].wait_send()
                send_bufs[q][s % 2] = part.astype(WIRE_DTYPE)
                if 1 <= s <= N_DEV - 3:
                    pl.semaphore_signal(
                        credits[q], inc=1,
                        device_id=(left if is_cw else right,),
                        device_id_type=pl.DeviceIdType.MESH)
                rdma = pltpu.make_async_remote_copy(
                    src_ref=send_bufs[q].at[s % 2],
                    dst_ref=recv_bufs[q].at[s % 2],
                    send_sem=ssems[q].at[s % 2],
                    recv_sem=rsems[q].at[s % 2],
                    device_id=(right if is_cw else left,),
                    device_id_type=pl.DeviceIdType.MESH)
                rdma.start()
                rdmas[q].append(rdma)

        scale = sx_ref[0] * sw_ref[0]
        last = (N_DEV - 2) % 2
        parts = [chunk_dot(p, col0) for _, col0 in rings]
        for q, (_, col0) in enumerate(rings):
            rdmas[q][N_DEV - 2].wait_recv()
            y = (parts[q] + recv_bufs[q][last].astype(jnp.float32)) * scale
            out_ref[:, pl.ds(col0, nq)] = (
                y * (1.0 / (1.0 + jnp.exp(-jnp.clip(y, -60.0, 60.0)))))

        for q in range(N_RINGS):
            rdmas[q][N_DEV - 3].wait_send()
            rdmas[q][N_DEV - 2].wait_send()

    return pl.pallas_call(
        body,
        out_shape=jax.ShapeDtypeStruct((m_per, n), jnp.float32),
        in_specs=[
            pl.BlockSpec(memory_space=pltpu.VMEM),
            pl.BlockSpec(memory_space=pltpu.VMEM),
            pl.BlockSpec(memory_space=pltpu.SMEM),
            pl.BlockSpec(memory_space=pltpu.SMEM),
        ],
        out_specs=pl.BlockSpec(memory_space=pltpu.VMEM),
        scratch_shapes=(
            [pltpu.VMEM((2, m_per, nq), WIRE_DTYPE) for _ in range(4)]
            + [pltpu.VMEM((2, m_per, nq), WIRE_DTYPE) for _ in range(4)]
            + [pltpu.SemaphoreType.DMA((2,)) for _ in range(8)]
            + [pltpu.SemaphoreType.REGULAR for _ in range(4)]
        ),
        compiler_params=pltpu.CompilerParams(collective_id=0),
    )(x, w_mat, scale_x, scale_w)


# device time: 355905 ns/iter; 2.1023x vs baseline; 1.0023x over previous
import jax
import jax.numpy as jnp
from jax import lax
from jax.experimental import pallas as pl
from jax.experimental.pallas import tpu as pltpu

N_DEV = 16
N_RINGS = 8
WIRE_DTYPE = jnp.bfloat16


def kernel(x, w_mat, scale_x, scale_w):
    m, k = x.shape
    _, n = w_mat.shape
    m_per = m // N_DEV
    nq = n // N_RINGS
    nr = N_RINGS

    rings = []
    for j in range(nr // 2):
        rings.append((True, j * nq))
        rings.append((False, (nr // 2 + j) * nq))

    def body(x_ref, w_ref, sx_ref, sw_ref, out_ref, *scratch):
        send_bufs = scratch[0:nr]
        recv_bufs = scratch[nr:2 * nr]
        ssems = scratch[2 * nr:3 * nr]
        rsems = scratch[3 * nr:4 * nr]
        credits = scratch[4 * nr:5 * nr]

        p = lax.axis_index("i")
        left = lax.rem(p - 1 + N_DEV, N_DEV)
        right = lax.rem(p + 1, N_DEV)

        barrier = pltpu.get_barrier_semaphore()
        for nbr in (left, right):
            pl.semaphore_signal(barrier, inc=1, device_id=(nbr,),
                                device_id_type=pl.DeviceIdType.MESH)
        pl.semaphore_wait(barrier, 2)

        def chunk_dot(c, col0):
            xc = x_ref[pl.ds(c * m_per, m_per), :].astype(jnp.bfloat16)
            wc = w_ref[:, pl.ds(col0, nq)].astype(jnp.bfloat16)
            return lax.dot_general(xc, wc, (((1,), (0,)), ((), ())),
                                   preferred_element_type=jnp.float32)

        rdmas = [[] for _ in rings]
        for s in range(N_DEV - 1):
            c_cw = lax.rem(p - 1 - s + 2 * N_DEV, N_DEV)
            c_ccw = lax.rem(p + 1 + s, N_DEV)
            parts = (None if s == 0 else
                     [chunk_dot(c_cw if is_cw else c_ccw, col0)
                      for is_cw, col0 in rings])
            for q, (is_cw, col0) in enumerate(rings):
                part = (chunk_dot(c_cw if is_cw else c_ccw, col0)
                        if s == 0 else parts[q])
                if s > 0:
                    rdmas[q][s - 1].wait_recv()
                    part = part + recv_bufs[q][(s - 1) % 2].astype(jnp.float32)
                if s >= 2:
                    pl.semaphore_wait(credits[q], 1)
                    rdmas[q][s - 2].wait_send()
                send_bufs[q][s % 2] = part.astype(WIRE_DTYPE)
                if 1 <= s <= N_DEV - 3:
                    pl.semaphore_signal(
                        credits[q], inc=1,
                        device_id=(left if is_cw else right,),
                        device_id_type=pl.DeviceIdType.MESH)
                rdma = pltpu.make_async_remote_copy(
                    src_ref=send_bufs[q].at[s % 2],
                    dst_ref=recv_bufs[q].at[s % 2],
                    send_sem=ssems[q].at[s % 2],
                    recv_sem=rsems[q].at[s % 2],
                    device_id=(right if is_cw else left,),
                    device_id_type=pl.DeviceIdType.MESH)
                rdma.start()
                rdmas[q].append(rdma)

        scale = sx_ref[0] * sw_ref[0]
        last = (N_DEV - 2) % 2
        parts = [chunk_dot(p, col0) for _, col0 in rings]
        for q, (_, col0) in enumerate(rings):
            rdmas[q][N_DEV - 2].wait_recv()
            y = (parts[q] + recv_bufs[q][last].astype(jnp.float32)) * scale
            out_ref[:, pl.ds(col0, nq)] = (
                y * (1.0 / (1.0 + jnp.exp(-jnp.clip(y, -60.0, 60.0)))))

        for q in range(N_RINGS):
            rdmas[q][N_DEV - 3].wait_send()
            rdmas[q][N_DEV - 2].wait_send()

    return pl.pallas_call(
        body,
        out_shape=jax.ShapeDtypeStruct((m_per, n), jnp.float32),
        in_specs=[
            pl.BlockSpec(memory_space=pltpu.VMEM),
            pl.BlockSpec(memory_space=pltpu.VMEM),
            pl.BlockSpec(memory_space=pltpu.SMEM),
            pl.BlockSpec(memory_space=pltpu.SMEM),
        ],
        out_specs=pl.BlockSpec(memory_space=pltpu.VMEM),
        scratch_shapes=(
            [pltpu.VMEM((2, m_per, nq), WIRE_DTYPE) for _ in range(nr)]
            + [pltpu.VMEM((2, m_per, nq), WIRE_DTYPE) for _ in range(nr)]
            + [pltpu.SemaphoreType.DMA((2,)) for _ in range(2 * nr)]
            + [pltpu.SemaphoreType.REGULAR for _ in range(nr)]
        ),
        compiler_params=pltpu.CompilerParams(collective_id=0),
    )(x, w_mat, scale_x, scale_w)
